# baseline (speedup 1.0000x reference)
import functools
import math

import jax
import jax.numpy as jnp
from jax import lax
from jax.experimental import pallas as pl
from jax.experimental.pallas import tpu as pltpu, tpu_sc as plsc

N = 10000
E = 320000
NP = 10240
ZROW = N
NS = 16
NTILES = 32
CH = 80
HALF = CH // 2
EPT = CH * 128
EP = NTILES * EPT
RPT = NP // NS
BLK = 1024
GRID = NP // BLK
LOG2PI = math.log(math.pi) / math.log(2.0)

_mesh = lambda: plsc.VectorSubcoreMesh(core_axis_name="c", subcore_axis_name="s",
                                       num_cores=2, num_subcores=NS)


@functools.cache
def _build_sc_deg():
    @functools.partial(
        pl.kernel,
        out_type=jax.ShapeDtypeStruct((2, NP, 128), jnp.float32),
        mesh=_mesh(),
        scratch_types=[
            pltpu.VMEM((CH, 128), jnp.int32),
            pltpu.VMEM((128, 128), jnp.float32),
            pltpu.VMEM_SHARED((NP, 128), jnp.float32),
            pltpu.SemaphoreType.DMA,
        ],
    )
    def _sc_deg(dst_hbm, zeros_hbm, ones_hbm, out_hbm, dst_v, ones_v, table, sem):
        c = lax.axis_index("c")
        s = lax.axis_index("s")
        wid = c * NS + s
        r0 = s * RPT
        pltpu.sync_copy(zeros_hbm.at[pl.ds(r0, RPT)], table.at[pl.ds(r0, RPT)])
        pltpu.sync_copy(ones_hbm, ones_v)
        pltpu.sync_copy(dst_hbm.at[wid], dst_v)
        plsc.subcore_barrier()

        W = 8

        def fire(j, carry):
            pltpu.async_copy(ones_v, table.at[dst_v.at[j]], sem, add=True)
            return carry

        def fire_drain(j, carry):
            pltpu.async_copy(ones_v, table.at[dst_v.at[j]], sem, add=True)
            pltpu.make_async_copy(ones_v, table.at[dst_v.at[j - W]], sem).wait()
            return carry

        def drain(j, carry):
            pltpu.make_async_copy(ones_v, table.at[dst_v.at[j]], sem).wait()
            return carry

        lax.fori_loop(0, W, fire, 0)
        lax.fori_loop(W, CH, fire_drain, 0)
        lax.fori_loop(CH - W, CH, drain, 0)
        plsc.subcore_barrier()
        pltpu.sync_copy(table.at[pl.ds(r0, RPT)], out_hbm.at[c, pl.ds(r0, RPT)])

    return _sc_deg


@functools.cache
def _build_sc_prop():
    @functools.partial(
        pl.kernel,
        out_type=jax.ShapeDtypeStruct((2, NP, 128), jnp.float32),
        mesh=_mesh(),
        scratch_types=[
            pltpu.VMEM((HALF, 128), jnp.int32),
            pltpu.VMEM((HALF, 128), jnp.int32),
            pltpu.VMEM((128, 128), jnp.float32),
            pltpu.VMEM((128, 128), jnp.float32),
            pltpu.VMEM_SHARED((NP, 128), jnp.float32),
            pltpu.SemaphoreType.DMA,
            pltpu.SemaphoreType.DMA,
            pltpu.SemaphoreType.DMA,
            pltpu.SemaphoreType.DMA,
        ],
    )
    def _sc_prop(src_hbm, dst_hbm, h_hbm, zeros_hbm, out_hbm,
                 src_v, dst_v, rows_a, rows_b, table, sem_a, sem_b,
                 ssem_a, ssem_b):
        c = lax.axis_index("c")
        s = lax.axis_index("s")
        wid = c * NS + s
        r0 = s * RPT
        pltpu.sync_copy(zeros_hbm.at[pl.ds(r0, RPT)], table.at[pl.ds(r0, RPT)])
        plsc.subcore_barrier()

        def gath(j, buf, sem):
            pltpu.async_copy(h_hbm.at[src_v.at[j]], buf, sem)

        def gwait(j, buf, sem):
            pltpu.make_async_copy(h_hbm.at[src_v.at[j]], buf, sem).wait()

        def scat(j, buf, sem):
            pltpu.async_copy(buf, table.at[dst_v.at[j]], sem, add=True)

        def swait(j, buf, sem):
            pltpu.make_async_copy(buf, table.at[dst_v.at[j]], sem).wait()

        for hp in range(2):
            pltpu.sync_copy(src_hbm.at[wid, pl.ds(hp * HALF, HALF)], src_v)
            pltpu.sync_copy(dst_hbm.at[wid, pl.ds(hp * HALF, HALF)], dst_v)
            gath(0, rows_a, sem_a)
            gath(1, rows_b, sem_b)

            def body(p, carry):
                j = 2 * p
                gwait(j, rows_a, sem_a)
                scat(j, rows_a, ssem_a)
                gwait(j + 1, rows_b, sem_b)
                scat(j + 1, rows_b, ssem_b)
                swait(j, rows_a, ssem_a)
                gath(j + 2, rows_a, sem_a)
                swait(j + 1, rows_b, ssem_b)
                gath(j + 3, rows_b, sem_b)
                return carry

            lax.fori_loop(0, HALF // 2 - 1, body, 0)
            gwait(HALF - 2, rows_a, sem_a)
            scat(HALF - 2, rows_a, ssem_a)
            gwait(HALF - 1, rows_b, sem_b)
            scat(HALF - 1, rows_b, ssem_b)
            swait(HALF - 2, rows_a, ssem_a)
            swait(HALF - 1, rows_b, ssem_b)
        plsc.subcore_barrier()
        pltpu.sync_copy(table.at[pl.ds(r0, RPT)], out_hbm.at[c, pl.ds(r0, RPT)])

    return _sc_prop


def _sc_deg(*args):
    return _build_sc_deg()(*args)


def _sc_prop(*args):
    return _build_sc_prop()(*args)


def _rowmask(i):
    rows = lax.broadcasted_iota(jnp.int32, (BLK, 1), 0) + i * BLK
    return rows < N


def _dot(a, b):
    return jnp.dot(a, b, preferred_element_type=jnp.float32,
                   precision=lax.Precision.HIGHEST)


def _tca_body(x_ref, degp_ref, wg_ref, bg_ref, h0_ref, hs0_ref, norm_ref):
    i = pl.program_id(0)
    deg = degp_ref[0, :, 0:1] + degp_ref[1, :, 0:1]
    norm = lax.rsqrt(jnp.maximum(deg, 1.0))
    h = jax.nn.sigmoid(_dot(x_ref[...], wg_ref[...]) + bg_ref[...])
    h = jnp.where(_rowmask(i), h, 0.0)
    h0_ref[...] = h
    hs0_ref[...] = h * norm
    norm_ref[...] = jnp.broadcast_to(norm, (BLK, 128))


def _tcscale_body(norm_ref, ap_ref, f_ref, fs_ref):
    norm = norm_ref[...]
    a = ap_ref[0] + ap_ref[1]
    f = a * norm
    f_ref[...] = f
    fs_ref[...] = f * norm


def _tcconv_body(norm_ref, ap_ref, h_ref, f1_ref, w_ref, b_ref, t_ref, ts_ref):
    i = pl.program_id(0)
    norm = norm_ref[...]
    f2 = (ap_ref[0] + ap_ref[1]) * norm
    w = w_ref[...]
    t = _dot(h_ref[...], w[0:128]) + _dot(f1_ref[...], w[128:256]) \
        + _dot(f2, w[256:384]) + b_ref[...]
    t = jnp.maximum(t, 0.0)
    t = jnp.where(_rowmask(i), t, 0.0)
    t_ref[...] = t
    ts_ref[...] = t * norm


def _tcfinal_body(norm_ref, ap_ref, t1_ref, g1_ref, wc_ref, bc_ref,
                  wd_ref, bd_ref, we1_ref, be1_ref, we21_ref, be21_ref,
                  we22_ref, be22_ref, wdc1_ref, bdc1_ref, wdc21_ref,
                  bdc21_ref, wdc22_ref, bdc22_ref, eps1_ref, eps2_ref,
                  zsum_ref, zest_ref, lrec_ref, kld_ref):
    i = pl.program_id(0)
    norm = norm_ref[...]
    g2 = (ap_ref[0] + ap_ref[1]) * norm
    wc = wc_ref[...]
    t2 = _dot(t1_ref[...], wc[0:128]) + _dot(g1_ref[...], wc[128:256]) \
        + _dot(g2, wc[256:384]) + bc_ref[...]
    t2 = jnp.maximum(t2, 0.0)
    hfin = _dot(t2, wd_ref[...]) + bd_ref[...]
    hfin = jnp.where(_rowmask(i), hfin, 0.0)

    @pl.when(i == 0)
    def _():
        zsum_ref[...] = jnp.zeros_like(zsum_ref)

    zsum_ref[...] += jnp.sum(hfin, axis=0, keepdims=True)

    @pl.when(i == GRID - 1)
    def _():
        z = zsum_ref[...] * (1.0 / N)
        h1 = jnp.maximum(_dot(z, we1_ref[...]) + be1_ref[...], 0.0)
        mu = _dot(h1, we21_ref[...]) + be21_ref[...]
        logvar = _dot(h1, we22_ref[...]) + be22_ref[...]
        latent = mu + eps1_ref[...] * jnp.exp(0.5 * logvar)
        h2 = jnp.maximum(_dot(latent, wdc1_ref[...]) + bdc1_ref[...], 0.0)
        z_mu = _dot(h2, wdc21_ref[...]) + bdc21_ref[...]
        z_logvar = _dot(h2, wdc22_ref[...]) + bdc22_ref[...]
        zest_ref[...] = z_mu + eps2_ref[...] * jnp.exp(0.5 * z_logvar)
        lrec_ref[...] = (LOG2PI + z_logvar
                         + (z - z_mu) ** 2 / (2.0 * jnp.exp(z_logvar)))
        kld_ref[...] = jnp.full((1, 128), -0.5 * jnp.sum(
            1.0 + logvar - mu ** 2 - jnp.exp(logvar)))


def _blk(ix=None):
    if ix is None:
        return pl.BlockSpec((BLK, 128), lambda i: (i, 0))
    return pl.BlockSpec(ix[0], ix[1])


_DEGP_SPEC = pl.BlockSpec((2, BLK, 128), lambda i: (0, i, 0))
_AP_SPEC = pl.BlockSpec((2, BLK, 128), lambda i: (0, i, 0))
_W128_SPEC = pl.BlockSpec((128, 128), lambda i: (0, 0))
_W384_SPEC = pl.BlockSpec((384, 128), lambda i: (0, 0))
_B_SPEC = pl.BlockSpec((1, 128), lambda i: (0, 0))
_NP_F32 = jax.ShapeDtypeStruct((NP, 128), jnp.float32)


def _tc_a(x, degp, wg, bg):
    return pl.pallas_call(
        _tca_body,
        grid=(GRID,),
        in_specs=[_blk(), _DEGP_SPEC, _W128_SPEC, _B_SPEC],
        out_specs=[_blk(), _blk(), _blk()],
        out_shape=[_NP_F32, _NP_F32, _NP_F32],
    )(x, degp, wg, bg)


def _tc_scale(norm, ap):
    return pl.pallas_call(
        _tcscale_body,
        grid=(GRID,),
        in_specs=[_blk(), _AP_SPEC],
        out_specs=[_blk(), _blk()],
        out_shape=[_NP_F32, _NP_F32],
    )(norm, ap)


def _tc_conv(norm, ap, h, f1, w, b):
    return pl.pallas_call(
        _tcconv_body,
        grid=(GRID,),
        in_specs=[_blk(), _AP_SPEC, _blk(), _blk(), _W384_SPEC, _B_SPEC],
        out_specs=[_blk(), _blk()],
        out_shape=[_NP_F32, _NP_F32],
    )(norm, ap, h, f1, w, b)


def _tc_final(norm, ap, t1, g1, wc, bc, wd, bd, *vae_ws):
    return pl.pallas_call(
        _tcfinal_body,
        grid=(GRID,),
        in_specs=[_blk(), _AP_SPEC, _blk(), _blk(), _W384_SPEC, _B_SPEC,
                  _W128_SPEC, _B_SPEC]
                 + [_W128_SPEC, _B_SPEC] * 6 + [_B_SPEC, _B_SPEC],
        out_specs=[pl.BlockSpec((1, 128), lambda i: (0, 0))] * 4,
        out_shape=[jax.ShapeDtypeStruct((1, 128), jnp.float32)] * 4,
    )(norm, ap, t1, g1, wc, bc, wd, bd, *vae_ws)


def _pad_w(w, b, fin, fout):
    wp = jnp.zeros((128, 128), jnp.float32).at[:fin, :fout].set(w)
    bp = jnp.zeros((1, 128), jnp.float32).at[0, :fout].set(b)
    return wp, bp


def kernel(x, edge_index, W_gate, b_gate, W_c0, b_c0, W_c1, b_c1, W_d, b_d,
           W_e1, b_e1, W_e21, b_e21, W_e22, b_e22,
           W_dc1, b_dc1, W_dc21, b_dc21, W_dc22, b_dc22):
    f32 = jnp.float32
    pad = ZROW + jnp.arange(EP - E, dtype=jnp.int32) % (NP - N)
    src_p = jnp.concatenate([edge_index[0], pad]).reshape(NTILES, CH, 128)
    dst_p = jnp.concatenate([edge_index[1], pad]).reshape(NTILES, CH, 128)
    zeros128 = jnp.zeros((NP, 128), f32)
    ones128 = jnp.ones((128, 128), f32)

    degp = _sc_deg(dst_p, zeros128, ones128)
    h0, hs0, norm = _tc_a(x, degp, W_gate, b_gate.reshape(1, 128))
    a1p = _sc_prop(src_p, dst_p, hs0, zeros128)
    f1, f1s = _tc_scale(norm, a1p)
    a2p = _sc_prop(src_p, dst_p, f1s, zeros128)
    t1, t1s = _tc_conv(norm, a2p, h0, f1, W_c0, b_c0.reshape(1, 128))
    b1p = _sc_prop(src_p, dst_p, t1s, zeros128)
    g1, g1s = _tc_scale(norm, b1p)
    b2p = _sc_prop(src_p, dst_p, g1s, zeros128)

    we1p, be1p = _pad_w(W_e1, b_e1, 128, 64)
    we21p, be21p = _pad_w(W_e21, b_e21, 64, 32)
    we22p, be22p = _pad_w(W_e22, b_e22, 64, 32)
    wdc1p, bdc1p = _pad_w(W_dc1, b_dc1, 32, 64)
    wdc21p, bdc21p = _pad_w(W_dc21, b_dc21, 64, 128)
    wdc22p, bdc22p = _pad_w(W_dc22, b_dc22, 64, 128)
    eps1 = jnp.zeros((1, 128), f32).at[0, :32].set(
        jax.random.normal(jax.random.key(1), (32,), f32))
    eps2 = jax.random.normal(jax.random.key(2), (128,), f32).reshape(1, 128)

    _, zest, lrec, kld = _tc_final(
        norm, b2p, t1, g1, W_c1, b_c1.reshape(1, 128),
        W_d, b_d.reshape(1, 128),
        we1p, be1p, we21p, be21p, we22p, be22p,
        wdc1p, bdc1p, wdc21p, bdc21p, wdc22p, bdc22p, eps1, eps2)
    return (zest.reshape(128), lrec.reshape(128), kld[0, 0])


# speedup vs baseline: 1.2233x; 1.2233x over previous
import functools
import math

import jax
import jax.numpy as jnp
from jax import lax
from jax.experimental import pallas as pl
from jax.experimental.pallas import tpu as pltpu, tpu_sc as plsc

N = 10000
E = 320000
NP = 10240
ZROW = N
NS = 16
NTILES = 32
CH = 80
HALF = CH // 2
EPT = CH * 128
EP = NTILES * EPT
RPT = NP // NS
BLK = 1024
GRID = NP // BLK
LOG2PI = math.log(math.pi) / math.log(2.0)

_mesh = lambda: plsc.VectorSubcoreMesh(core_axis_name="c", subcore_axis_name="s",
                                       num_cores=2, num_subcores=NS)


@functools.cache
def _build_sc_deg():
    @functools.partial(
        pl.kernel,
        out_type=jax.ShapeDtypeStruct((2, NP, 128), jnp.float32),
        mesh=_mesh(),
        scratch_types=[
            pltpu.VMEM((CH, 128), jnp.int32),
            pltpu.VMEM((128, 128), jnp.float32),
            pltpu.VMEM_SHARED((NP, 128), jnp.float32),
            pltpu.SemaphoreType.DMA,
        ],
    )
    def _sc_deg(dst_hbm, zeros_hbm, ones_hbm, out_hbm, dst_v, ones_v, table, sem):
        c = lax.axis_index("c")
        s = lax.axis_index("s")
        wid = c * NS + s
        r0 = s * RPT
        pltpu.sync_copy(zeros_hbm.at[pl.ds(r0, RPT)], table.at[pl.ds(r0, RPT)])
        pltpu.sync_copy(ones_hbm, ones_v)
        pltpu.sync_copy(dst_hbm.at[wid], dst_v)
        plsc.subcore_barrier()

        W = 8

        def fire(j, carry):
            pltpu.async_copy(ones_v, table.at[dst_v.at[j]], sem, add=True)
            return carry

        def fire_drain(j, carry):
            pltpu.async_copy(ones_v, table.at[dst_v.at[j]], sem, add=True)
            pltpu.make_async_copy(ones_v, table.at[dst_v.at[j - W]], sem).wait()
            return carry

        def drain(j, carry):
            pltpu.make_async_copy(ones_v, table.at[dst_v.at[j]], sem).wait()
            return carry

        lax.fori_loop(0, W, fire, 0)
        lax.fori_loop(W, CH, fire_drain, 0)
        lax.fori_loop(CH - W, CH, drain, 0)
        plsc.subcore_barrier()
        pltpu.sync_copy(table.at[pl.ds(r0, RPT)], out_hbm.at[c, pl.ds(r0, RPT)])

    return _sc_deg


@functools.cache
def _build_sc_prop():
    @functools.partial(
        pl.kernel,
        out_type=jax.ShapeDtypeStruct((2, NP, 128), jnp.float32),
        mesh=_mesh(),
        scratch_types=[
            pltpu.VMEM((HALF, 128), jnp.int32),
            pltpu.VMEM((HALF, 128), jnp.int32),
            pltpu.VMEM((128, 128), jnp.float32),
            pltpu.VMEM((128, 128), jnp.float32),
            pltpu.VMEM_SHARED((NP, 128), jnp.float32),
            pltpu.SemaphoreType.DMA,
            pltpu.SemaphoreType.DMA,
        ],
    )
    def _sc_prop(src_hbm, dst_hbm, h_hbm, zeros_hbm, out_hbm,
                 src_v, dst_v, rows_a, rows_b, table, sem_a, sem_b):
        c = lax.axis_index("c")
        s = lax.axis_index("s")
        wid = c * NS + s
        r0 = s * RPT
        pltpu.sync_copy(zeros_hbm.at[pl.ds(r0, RPT)], table.at[pl.ds(r0, RPT)])
        plsc.subcore_barrier()

        def gath(j, buf, sem):
            pltpu.async_copy(h_hbm.at[src_v.at[j]], buf, sem)

        def scat(j, buf, sem):
            pltpu.make_async_copy(h_hbm.at[src_v.at[j]], buf, sem).wait()
            pltpu.sync_copy(buf, table.at[dst_v.at[j]], add=True)

        for hp in range(2):
            pltpu.sync_copy(src_hbm.at[wid, pl.ds(hp * HALF, HALF)], src_v)
            pltpu.sync_copy(dst_hbm.at[wid, pl.ds(hp * HALF, HALF)], dst_v)
            gath(0, rows_a, sem_a)
            gath(1, rows_b, sem_b)

            def body(p, carry):
                j = 2 * p
                scat(j, rows_a, sem_a)
                gath(j + 2, rows_a, sem_a)
                scat(j + 1, rows_b, sem_b)
                gath(j + 3, rows_b, sem_b)
                return carry

            lax.fori_loop(0, HALF // 2 - 1, body, 0)
            scat(HALF - 2, rows_a, sem_a)
            scat(HALF - 1, rows_b, sem_b)
        plsc.subcore_barrier()
        pltpu.sync_copy(table.at[pl.ds(r0, RPT)], out_hbm.at[c, pl.ds(r0, RPT)])

    return _sc_prop


def _sc_deg(*args):
    return _build_sc_deg()(*args)


def _sc_prop(*args):
    return _build_sc_prop()(*args)


def _rowmask(i):
    rows = lax.broadcasted_iota(jnp.int32, (BLK, 1), 0) + i * BLK
    return rows < N


def _dot(a, b):
    return jnp.dot(a, b, preferred_element_type=jnp.float32,
                   precision=lax.Precision.HIGHEST)


def _tca_body(x_ref, degp_ref, wg_ref, bg_ref, h0_ref, hs0_ref, norm_ref):
    i = pl.program_id(0)
    deg = degp_ref[0, :, 0:1] + degp_ref[1, :, 0:1]
    norm = lax.rsqrt(jnp.maximum(deg, 1.0))
    h = jax.nn.sigmoid(_dot(x_ref[...], wg_ref[...]) + bg_ref[...])
    h = jnp.where(_rowmask(i), h, 0.0)
    h0_ref[...] = h
    hs0_ref[...] = h * norm
    norm_ref[...] = jnp.broadcast_to(norm, (BLK, 128))


def _tcscale_body(norm_ref, ap_ref, f_ref, fs_ref):
    norm = norm_ref[...]
    a = ap_ref[0] + ap_ref[1]
    f = a * norm
    f_ref[...] = f
    fs_ref[...] = f * norm


def _tcconv_body(norm_ref, ap_ref, h_ref, f1_ref, w_ref, b_ref, t_ref, ts_ref):
    i = pl.program_id(0)
    norm = norm_ref[...]
    f2 = (ap_ref[0] + ap_ref[1]) * norm
    w = w_ref[...]
    t = _dot(h_ref[...], w[0:128]) + _dot(f1_ref[...], w[128:256]) \
        + _dot(f2, w[256:384]) + b_ref[...]
    t = jnp.maximum(t, 0.0)
    t = jnp.where(_rowmask(i), t, 0.0)
    t_ref[...] = t
    ts_ref[...] = t * norm


def _tcfinal_body(norm_ref, ap_ref, t1_ref, g1_ref, wc_ref, bc_ref,
                  wd_ref, bd_ref, we1_ref, be1_ref, we21_ref, be21_ref,
                  we22_ref, be22_ref, wdc1_ref, bdc1_ref, wdc21_ref,
                  bdc21_ref, wdc22_ref, bdc22_ref, eps1_ref, eps2_ref,
                  zsum_ref, zest_ref, lrec_ref, kld_ref):
    i = pl.program_id(0)
    norm = norm_ref[...]
    g2 = (ap_ref[0] + ap_ref[1]) * norm
    wc = wc_ref[...]
    t2 = _dot(t1_ref[...], wc[0:128]) + _dot(g1_ref[...], wc[128:256]) \
        + _dot(g2, wc[256:384]) + bc_ref[...]
    t2 = jnp.maximum(t2, 0.0)
    hfin = _dot(t2, wd_ref[...]) + bd_ref[...]
    hfin = jnp.where(_rowmask(i), hfin, 0.0)

    @pl.when(i == 0)
    def _():
        zsum_ref[...] = jnp.zeros_like(zsum_ref)

    zsum_ref[...] += jnp.sum(hfin, axis=0, keepdims=True)

    @pl.when(i == GRID - 1)
    def _():
        z = zsum_ref[...] * (1.0 / N)
        h1 = jnp.maximum(_dot(z, we1_ref[...]) + be1_ref[...], 0.0)
        mu = _dot(h1, we21_ref[...]) + be21_ref[...]
        logvar = _dot(h1, we22_ref[...]) + be22_ref[...]
        latent = mu + eps1_ref[...] * jnp.exp(0.5 * logvar)
        h2 = jnp.maximum(_dot(latent, wdc1_ref[...]) + bdc1_ref[...], 0.0)
        z_mu = _dot(h2, wdc21_ref[...]) + bdc21_ref[...]
        z_logvar = _dot(h2, wdc22_ref[...]) + bdc22_ref[...]
        zest_ref[...] = z_mu + eps2_ref[...] * jnp.exp(0.5 * z_logvar)
        lrec_ref[...] = (LOG2PI + z_logvar
                         + (z - z_mu) ** 2 / (2.0 * jnp.exp(z_logvar)))
        kld_ref[...] = jnp.full((1, 128), -0.5 * jnp.sum(
            1.0 + logvar - mu ** 2 - jnp.exp(logvar)))


def _blk(ix=None):
    if ix is None:
        return pl.BlockSpec((BLK, 128), lambda i: (i, 0))
    return pl.BlockSpec(ix[0], ix[1])


_DEGP_SPEC = pl.BlockSpec((2, BLK, 128), lambda i: (0, i, 0))
_AP_SPEC = pl.BlockSpec((2, BLK, 128), lambda i: (0, i, 0))
_W128_SPEC = pl.BlockSpec((128, 128), lambda i: (0, 0))
_W384_SPEC = pl.BlockSpec((384, 128), lambda i: (0, 0))
_B_SPEC = pl.BlockSpec((1, 128), lambda i: (0, 0))
_NP_F32 = jax.ShapeDtypeStruct((NP, 128), jnp.float32)


def _tc_a(x, degp, wg, bg):
    return pl.pallas_call(
        _tca_body,
        grid=(GRID,),
        in_specs=[_blk(), _DEGP_SPEC, _W128_SPEC, _B_SPEC],
        out_specs=[_blk(), _blk(), _blk()],
        out_shape=[_NP_F32, _NP_F32, _NP_F32],
    )(x, degp, wg, bg)


def _tc_scale(norm, ap):
    return pl.pallas_call(
        _tcscale_body,
        grid=(GRID,),
        in_specs=[_blk(), _AP_SPEC],
        out_specs=[_blk(), _blk()],
        out_shape=[_NP_F32, _NP_F32],
    )(norm, ap)


def _tc_conv(norm, ap, h, f1, w, b):
    return pl.pallas_call(
        _tcconv_body,
        grid=(GRID,),
        in_specs=[_blk(), _AP_SPEC, _blk(), _blk(), _W384_SPEC, _B_SPEC],
        out_specs=[_blk(), _blk()],
        out_shape=[_NP_F32, _NP_F32],
    )(norm, ap, h, f1, w, b)


def _tc_final(norm, ap, t1, g1, wc, bc, wd, bd, *vae_ws):
    return pl.pallas_call(
        _tcfinal_body,
        grid=(GRID,),
        in_specs=[_blk(), _AP_SPEC, _blk(), _blk(), _W384_SPEC, _B_SPEC,
                  _W128_SPEC, _B_SPEC]
                 + [_W128_SPEC, _B_SPEC] * 6 + [_B_SPEC, _B_SPEC],
        out_specs=[pl.BlockSpec((1, 128), lambda i: (0, 0))] * 4,
        out_shape=[jax.ShapeDtypeStruct((1, 128), jnp.float32)] * 4,
    )(norm, ap, t1, g1, wc, bc, wd, bd, *vae_ws)


def _pad_w(w, b, fin, fout):
    wp = jnp.zeros((128, 128), jnp.float32).at[:fin, :fout].set(w)
    bp = jnp.zeros((1, 128), jnp.float32).at[0, :fout].set(b)
    return wp, bp


def kernel(x, edge_index, W_gate, b_gate, W_c0, b_c0, W_c1, b_c1, W_d, b_d,
           W_e1, b_e1, W_e21, b_e21, W_e22, b_e22,
           W_dc1, b_dc1, W_dc21, b_dc21, W_dc22, b_dc22):
    f32 = jnp.float32
    pad = ZROW + jnp.arange(EP - E, dtype=jnp.int32) % (NP - N)
    src_p = jnp.concatenate([edge_index[0], pad]).reshape(NTILES, CH, 128)
    dst_p = jnp.concatenate([edge_index[1], pad]).reshape(NTILES, CH, 128)
    zeros128 = jnp.zeros((NP, 128), f32)
    ones128 = jnp.ones((128, 128), f32)

    degp = _sc_deg(dst_p, zeros128, ones128)
    h0, hs0, norm = _tc_a(x, degp, W_gate, b_gate.reshape(1, 128))
    a1p = _sc_prop(src_p, dst_p, hs0, zeros128)
    f1, f1s = _tc_scale(norm, a1p)
    a2p = _sc_prop(src_p, dst_p, f1s, zeros128)
    t1, t1s = _tc_conv(norm, a2p, h0, f1, W_c0, b_c0.reshape(1, 128))
    b1p = _sc_prop(src_p, dst_p, t1s, zeros128)
    g1, g1s = _tc_scale(norm, b1p)
    b2p = _sc_prop(src_p, dst_p, g1s, zeros128)

    we1p, be1p = _pad_w(W_e1, b_e1, 128, 64)
    we21p, be21p = _pad_w(W_e21, b_e21, 64, 32)
    we22p, be22p = _pad_w(W_e22, b_e22, 64, 32)
    wdc1p, bdc1p = _pad_w(W_dc1, b_dc1, 32, 64)
    wdc21p, bdc21p = _pad_w(W_dc21, b_dc21, 64, 128)
    wdc22p, bdc22p = _pad_w(W_dc22, b_dc22, 64, 128)
    eps1 = jnp.zeros((1, 128), f32).at[0, :32].set(
        jax.random.normal(jax.random.key(1), (32,), f32))
    eps2 = jax.random.normal(jax.random.key(2), (128,), f32).reshape(1, 128)

    _, zest, lrec, kld = _tc_final(
        norm, b2p, t1, g1, W_c1, b_c1.reshape(1, 128),
        W_d, b_d.reshape(1, 128),
        we1p, be1p, we21p, be21p, we22p, be22p,
        wdc1p, bdc1p, wdc21p, bdc21p, wdc22p, bdc22p, eps1, eps2)
    return (zest.reshape(128), lrec.reshape(128), kld[0, 0])


# speedup vs baseline: 1.2426x; 1.0158x over previous
import functools
import math

import jax
import jax.numpy as jnp
from jax import lax
from jax.experimental import pallas as pl
from jax.experimental.pallas import tpu as pltpu, tpu_sc as plsc

N = 10000
E = 320000
NP = 10240
ZROW = N
NS = 16
NTILES = 32
CH = 80
HALF = CH // 2
EPT = CH * 128
EP = NTILES * EPT
RPT = NP // NS
BLK = 1024
GRID = NP // BLK
LOG2PI = math.log(math.pi) / math.log(2.0)

_mesh = lambda: plsc.VectorSubcoreMesh(core_axis_name="c", subcore_axis_name="s",
                                       num_cores=2, num_subcores=NS)


@functools.cache
def _build_sc_deg():
    @functools.partial(
        pl.kernel,
        out_type=jax.ShapeDtypeStruct((2, NP, 128), jnp.float32),
        mesh=_mesh(),
        scratch_types=[
            pltpu.VMEM((CH, 128), jnp.int32),
            pltpu.VMEM((128, 128), jnp.float32),
            pltpu.VMEM_SHARED((NP, 128), jnp.float32),
            pltpu.SemaphoreType.DMA,
        ],
    )
    def _sc_deg(dst_hbm, zeros_hbm, ones_hbm, out_hbm, dst_v, ones_v, table, sem):
        c = lax.axis_index("c")
        s = lax.axis_index("s")
        wid = c * NS + s
        r0 = s * RPT
        pltpu.sync_copy(zeros_hbm.at[pl.ds(r0, RPT)], table.at[pl.ds(r0, RPT)])
        pltpu.sync_copy(ones_hbm, ones_v)
        pltpu.sync_copy(dst_hbm.at[wid], dst_v)
        plsc.subcore_barrier()

        W = 8

        def fire(j, carry):
            pltpu.async_copy(ones_v, table.at[dst_v.at[j]], sem, add=True)
            return carry

        def fire_drain(j, carry):
            pltpu.async_copy(ones_v, table.at[dst_v.at[j]], sem, add=True)
            pltpu.make_async_copy(ones_v, table.at[dst_v.at[j - W]], sem).wait()
            return carry

        def drain(j, carry):
            pltpu.make_async_copy(ones_v, table.at[dst_v.at[j]], sem).wait()
            return carry

        lax.fori_loop(0, W, fire, 0)
        lax.fori_loop(W, CH, fire_drain, 0)
        lax.fori_loop(CH - W, CH, drain, 0)
        plsc.subcore_barrier()
        pltpu.sync_copy(table.at[pl.ds(r0, RPT)], out_hbm.at[c, pl.ds(r0, RPT)])

    return _sc_deg


@functools.cache
def _build_sc_prop():
    @functools.partial(
        pl.kernel,
        out_type=jax.ShapeDtypeStruct((2, NP, 128), jnp.float32),
        mesh=_mesh(),
        scratch_types=[
            pltpu.VMEM((HALF, 128), jnp.int32),
            pltpu.VMEM((HALF, 128), jnp.int32),
            pltpu.VMEM((128, 128), jnp.float32),
            pltpu.VMEM((128, 128), jnp.float32),
            pltpu.VMEM_SHARED((NP, 128), jnp.float32),
            pltpu.SemaphoreType.DMA,
            pltpu.SemaphoreType.DMA,
        ],
    )
    def _sc_prop(src_hbm, dst_hbm, h_hbm, zeros_hbm, out_hbm,
                 src_v, dst_v, rows_a, rows_b, table, sem_a, sem_b):
        c = lax.axis_index("c")
        s = lax.axis_index("s")
        wid = c * NS + s
        r0 = s * RPT

        def gath(j, buf, sem):
            pltpu.async_copy(h_hbm.at[src_v.at[j]], buf, sem)

        def scat(j, buf, sem):
            pltpu.make_async_copy(h_hbm.at[src_v.at[j]], buf, sem).wait()
            pltpu.sync_copy(buf, table.at[dst_v.at[j]], add=True)

        pltpu.sync_copy(src_hbm.at[wid, pl.ds(0, HALF)], src_v)
        pltpu.sync_copy(dst_hbm.at[wid, pl.ds(0, HALF)], dst_v)
        gath(0, rows_a, sem_a)
        gath(1, rows_b, sem_b)
        pltpu.sync_copy(zeros_hbm.at[pl.ds(r0, RPT)], table.at[pl.ds(r0, RPT)])
        plsc.subcore_barrier()

        for hp in range(2):
            if hp:
                pltpu.sync_copy(src_hbm.at[wid, pl.ds(hp * HALF, HALF)], src_v)
                pltpu.sync_copy(dst_hbm.at[wid, pl.ds(hp * HALF, HALF)], dst_v)
                gath(0, rows_a, sem_a)
                gath(1, rows_b, sem_b)

            def body(p, carry):
                j = 2 * p
                scat(j, rows_a, sem_a)
                gath(j + 2, rows_a, sem_a)
                scat(j + 1, rows_b, sem_b)
                gath(j + 3, rows_b, sem_b)
                return carry

            lax.fori_loop(0, HALF // 2 - 1, body, 0)
            scat(HALF - 2, rows_a, sem_a)
            scat(HALF - 1, rows_b, sem_b)
        plsc.subcore_barrier()
        pltpu.sync_copy(table.at[pl.ds(r0, RPT)], out_hbm.at[c, pl.ds(r0, RPT)])

    return _sc_prop


def _sc_deg(*args):
    return _build_sc_deg()(*args)


def _sc_prop(*args):
    return _build_sc_prop()(*args)


def _rowmask(i):
    rows = lax.broadcasted_iota(jnp.int32, (BLK, 1), 0) + i * BLK
    return rows < N


def _dot(a, b):
    return jnp.dot(a, b, preferred_element_type=jnp.float32,
                   precision=lax.Precision.HIGHEST)


def _tcgate_body(x_ref, wg_ref, bg_ref, h0_ref):
    i = pl.program_id(0)
    h = jax.nn.sigmoid(_dot(x_ref[...], wg_ref[...]) + bg_ref[...])
    h0_ref[...] = jnp.where(_rowmask(i), h, 0.0)


def _tcnorm_body(degp_ref, h0_ref, hs0_ref, norm_ref):
    deg = degp_ref[0, :, 0:1] + degp_ref[1, :, 0:1]
    norm = lax.rsqrt(jnp.maximum(deg, 1.0))
    hs0_ref[...] = h0_ref[...] * norm
    norm_ref[...] = jnp.broadcast_to(norm, (BLK, 128))


def _tcscale_body(norm_ref, ap_ref, f_ref, fs_ref):
    norm = norm_ref[...]
    a = ap_ref[0] + ap_ref[1]
    f = a * norm
    f_ref[...] = f
    fs_ref[...] = f * norm


def _tcconv_body(norm_ref, ap_ref, h_ref, f1_ref, w_ref, b_ref, t_ref, ts_ref):
    i = pl.program_id(0)
    norm = norm_ref[...]
    f2 = (ap_ref[0] + ap_ref[1]) * norm
    w = w_ref[...]
    t = _dot(h_ref[...], w[0:128]) + _dot(f1_ref[...], w[128:256]) \
        + _dot(f2, w[256:384]) + b_ref[...]
    t = jnp.maximum(t, 0.0)
    t = jnp.where(_rowmask(i), t, 0.0)
    t_ref[...] = t
    ts_ref[...] = t * norm


def _tcfinal_body(norm_ref, ap_ref, t1_ref, g1_ref, wc_ref, bc_ref,
                  wd_ref, bd_ref, we1_ref, be1_ref, we21_ref, be21_ref,
                  we22_ref, be22_ref, wdc1_ref, bdc1_ref, wdc21_ref,
                  bdc21_ref, wdc22_ref, bdc22_ref, eps1_ref, eps2_ref,
                  zsum_ref, zest_ref, lrec_ref, kld_ref):
    i = pl.program_id(0)
    norm = norm_ref[...]
    g2 = (ap_ref[0] + ap_ref[1]) * norm
    wc = wc_ref[...]
    t2 = _dot(t1_ref[...], wc[0:128]) + _dot(g1_ref[...], wc[128:256]) \
        + _dot(g2, wc[256:384]) + bc_ref[...]
    t2 = jnp.maximum(t2, 0.0)
    hfin = _dot(t2, wd_ref[...]) + bd_ref[...]
    hfin = jnp.where(_rowmask(i), hfin, 0.0)

    @pl.when(i == 0)
    def _():
        zsum_ref[...] = jnp.zeros_like(zsum_ref)

    zsum_ref[...] += jnp.sum(hfin, axis=0, keepdims=True)

    @pl.when(i == GRID - 1)
    def _():
        z = zsum_ref[...] * (1.0 / N)
        h1 = jnp.maximum(_dot(z, we1_ref[...]) + be1_ref[...], 0.0)
        mu = _dot(h1, we21_ref[...]) + be21_ref[...]
        logvar = _dot(h1, we22_ref[...]) + be22_ref[...]
        latent = mu + eps1_ref[...] * jnp.exp(0.5 * logvar)
        h2 = jnp.maximum(_dot(latent, wdc1_ref[...]) + bdc1_ref[...], 0.0)
        z_mu = _dot(h2, wdc21_ref[...]) + bdc21_ref[...]
        z_logvar = _dot(h2, wdc22_ref[...]) + bdc22_ref[...]
        zest_ref[...] = z_mu + eps2_ref[...] * jnp.exp(0.5 * z_logvar)
        lrec_ref[...] = (LOG2PI + z_logvar
                         + (z - z_mu) ** 2 / (2.0 * jnp.exp(z_logvar)))
        kld_ref[...] = jnp.full((1, 128), -0.5 * jnp.sum(
            1.0 + logvar - mu ** 2 - jnp.exp(logvar)))


def _blk(ix=None):
    if ix is None:
        return pl.BlockSpec((BLK, 128), lambda i: (i, 0))
    return pl.BlockSpec(ix[0], ix[1])


_DEGP_SPEC = pl.BlockSpec((2, BLK, 128), lambda i: (0, i, 0))
_AP_SPEC = pl.BlockSpec((2, BLK, 128), lambda i: (0, i, 0))
_W128_SPEC = pl.BlockSpec((128, 128), lambda i: (0, 0))
_W384_SPEC = pl.BlockSpec((384, 128), lambda i: (0, 0))
_B_SPEC = pl.BlockSpec((1, 128), lambda i: (0, 0))
_NP_F32 = jax.ShapeDtypeStruct((NP, 128), jnp.float32)


def _tc_gate(x, wg, bg):
    return pl.pallas_call(
        _tcgate_body,
        grid=(GRID,),
        in_specs=[_blk(), _W128_SPEC, _B_SPEC],
        out_specs=_blk(),
        out_shape=_NP_F32,
    )(x, wg, bg)


def _tc_norm(degp, h0):
    return pl.pallas_call(
        _tcnorm_body,
        grid=(GRID,),
        in_specs=[_DEGP_SPEC, _blk()],
        out_specs=[_blk(), _blk()],
        out_shape=[_NP_F32, _NP_F32],
    )(degp, h0)


def _tc_scale(norm, ap):
    return pl.pallas_call(
        _tcscale_body,
        grid=(GRID,),
        in_specs=[_blk(), _AP_SPEC],
        out_specs=[_blk(), _blk()],
        out_shape=[_NP_F32, _NP_F32],
    )(norm, ap)


def _tc_conv(norm, ap, h, f1, w, b):
    return pl.pallas_call(
        _tcconv_body,
        grid=(GRID,),
        in_specs=[_blk(), _AP_SPEC, _blk(), _blk(), _W384_SPEC, _B_SPEC],
        out_specs=[_blk(), _blk()],
        out_shape=[_NP_F32, _NP_F32],
    )(norm, ap, h, f1, w, b)


def _tc_final(norm, ap, t1, g1, wc, bc, wd, bd, *vae_ws):
    return pl.pallas_call(
        _tcfinal_body,
        grid=(GRID,),
        in_specs=[_blk(), _AP_SPEC, _blk(), _blk(), _W384_SPEC, _B_SPEC,
                  _W128_SPEC, _B_SPEC]
                 + [_W128_SPEC, _B_SPEC] * 6 + [_B_SPEC, _B_SPEC],
        out_specs=[pl.BlockSpec((1, 128), lambda i: (0, 0))] * 4,
        out_shape=[jax.ShapeDtypeStruct((1, 128), jnp.float32)] * 4,
    )(norm, ap, t1, g1, wc, bc, wd, bd, *vae_ws)


def _pad_w(w, b, fin, fout):
    wp = jnp.zeros((128, 128), jnp.float32).at[:fin, :fout].set(w)
    bp = jnp.zeros((1, 128), jnp.float32).at[0, :fout].set(b)
    return wp, bp


def kernel(x, edge_index, W_gate, b_gate, W_c0, b_c0, W_c1, b_c1, W_d, b_d,
           W_e1, b_e1, W_e21, b_e21, W_e22, b_e22,
           W_dc1, b_dc1, W_dc21, b_dc21, W_dc22, b_dc22):
    f32 = jnp.float32
    pad = ZROW + jnp.arange(EP - E, dtype=jnp.int32) % (NP - N)
    src_p = jnp.concatenate([edge_index[0], pad]).reshape(NTILES, CH, 128)
    dst_p = jnp.concatenate([edge_index[1], pad]).reshape(NTILES, CH, 128)
    zeros128 = jnp.zeros((NP, 128), f32)
    ones128 = jnp.ones((128, 128), f32)

    degp = _sc_deg(dst_p, zeros128, ones128)
    h0 = _tc_gate(x, W_gate, b_gate.reshape(1, 128))
    hs0, norm = _tc_norm(degp, h0)
    a1p = _sc_prop(src_p, dst_p, hs0, zeros128)
    f1, f1s = _tc_scale(norm, a1p)
    a2p = _sc_prop(src_p, dst_p, f1s, zeros128)
    t1, t1s = _tc_conv(norm, a2p, h0, f1, W_c0, b_c0.reshape(1, 128))
    b1p = _sc_prop(src_p, dst_p, t1s, zeros128)
    g1, g1s = _tc_scale(norm, b1p)
    b2p = _sc_prop(src_p, dst_p, g1s, zeros128)

    we1p, be1p = _pad_w(W_e1, b_e1, 128, 64)
    we21p, be21p = _pad_w(W_e21, b_e21, 64, 32)
    we22p, be22p = _pad_w(W_e22, b_e22, 64, 32)
    wdc1p, bdc1p = _pad_w(W_dc1, b_dc1, 32, 64)
    wdc21p, bdc21p = _pad_w(W_dc21, b_dc21, 64, 128)
    wdc22p, bdc22p = _pad_w(W_dc22, b_dc22, 64, 128)
    eps1 = jnp.zeros((1, 128), f32).at[0, :32].set(
        jax.random.normal(jax.random.key(1), (32,), f32))
    eps2 = jax.random.normal(jax.random.key(2), (128,), f32).reshape(1, 128)

    _, zest, lrec, kld = _tc_final(
        norm, b2p, t1, g1, W_c1, b_c1.reshape(1, 128),
        W_d, b_d.reshape(1, 128),
        we1p, be1p, we21p, be21p, we22p, be22p,
        wdc1p, bdc1p, wdc21p, bdc21p, wdc22p, bdc22p, eps1, eps2)
    return (zest.reshape(128), lrec.reshape(128), kld[0, 0])


# speedup vs baseline: 1.2645x; 1.0176x over previous
import functools
import math

import jax
import jax.numpy as jnp
from jax import lax
from jax.experimental import pallas as pl
from jax.experimental.pallas import tpu as pltpu, tpu_sc as plsc

N = 10000
E = 320000
NP = 10240
ZROW = N
NS = 16
NTILES = 32
CH = 80
HALF = CH // 2
EPT = CH * 128
EP = NTILES * EPT
RPT = NP // NS
BLK = 1024
GRID = NP // BLK
LOG2PI = math.log(math.pi) / math.log(2.0)

_mesh = lambda: plsc.VectorSubcoreMesh(core_axis_name="c", subcore_axis_name="s",
                                       num_cores=2, num_subcores=NS)


@functools.cache
def _build_sc_deg():
    @functools.partial(
        pl.kernel,
        out_type=jax.ShapeDtypeStruct((2, NP, 128), jnp.float32),
        mesh=_mesh(),
        scratch_types=[
            pltpu.VMEM((CH, 128), jnp.int32),
            pltpu.VMEM((128, 128), jnp.float32),
            pltpu.VMEM_SHARED((NP, 128), jnp.float32),
            pltpu.SemaphoreType.DMA,
        ],
    )
    def _sc_deg(dst_hbm, zeros_hbm, ones_hbm, out_hbm, dst_v, ones_v, table, sem):
        c = lax.axis_index("c")
        s = lax.axis_index("s")
        wid = c * NS + s
        r0 = s * RPT
        pltpu.sync_copy(zeros_hbm.at[pl.ds(r0, RPT)], table.at[pl.ds(r0, RPT)])
        pltpu.sync_copy(ones_hbm, ones_v)
        pltpu.sync_copy(dst_hbm.at[wid], dst_v)
        plsc.subcore_barrier()

        W = 8

        def fire(j, carry):
            pltpu.async_copy(ones_v, table.at[dst_v.at[j]], sem, add=True)
            return carry

        def fire_drain(j, carry):
            pltpu.async_copy(ones_v, table.at[dst_v.at[j]], sem, add=True)
            pltpu.make_async_copy(ones_v, table.at[dst_v.at[j - W]], sem).wait()
            return carry

        def drain(j, carry):
            pltpu.make_async_copy(ones_v, table.at[dst_v.at[j]], sem).wait()
            return carry

        lax.fori_loop(0, W, fire, 0)
        lax.fori_loop(W, CH, fire_drain, 0)
        lax.fori_loop(CH - W, CH, drain, 0)
        plsc.subcore_barrier()
        pltpu.sync_copy(table.at[pl.ds(r0, RPT)], out_hbm.at[c, pl.ds(r0, RPT)])

    return _sc_deg


@functools.cache
def _build_sc_prop():
    @functools.partial(
        pl.kernel,
        out_type=jax.ShapeDtypeStruct((2, NP, 128), jnp.float32),
        mesh=_mesh(),
        scratch_types=[
            pltpu.VMEM((HALF, 128), jnp.int32),
            pltpu.VMEM((HALF, 128), jnp.int32),
            pltpu.VMEM((128, 128), jnp.float32),
            pltpu.VMEM((128, 128), jnp.float32),
            pltpu.VMEM_SHARED((NP, 128), jnp.float32),
            pltpu.SemaphoreType.DMA,
            pltpu.SemaphoreType.DMA,
        ],
    )
    def _sc_prop(src_hbm, dst_hbm, h_hbm, zeros_hbm, out_hbm,
                 src_v, dst_v, rows_a, rows_b, table, sem_a, sem_b):
        c = lax.axis_index("c")
        s = lax.axis_index("s")
        wid = c * NS + s
        r0 = s * RPT

        def gath(j, buf, sem):
            pltpu.async_copy(h_hbm.at[src_v.at[j]], buf, sem)

        def scat(j, buf, sem):
            pltpu.make_async_copy(h_hbm.at[src_v.at[j]], buf, sem).wait()
            pltpu.sync_copy(buf, table.at[dst_v.at[j]], add=True)

        pltpu.sync_copy(src_hbm.at[wid, pl.ds(0, HALF)], src_v)
        pltpu.sync_copy(dst_hbm.at[wid, pl.ds(0, HALF)], dst_v)
        gath(0, rows_a, sem_a)
        gath(1, rows_b, sem_b)
        pltpu.sync_copy(zeros_hbm.at[pl.ds(r0, RPT)], table.at[pl.ds(r0, RPT)])
        plsc.subcore_barrier()

        for hp in range(2):
            if hp:
                pltpu.sync_copy(src_hbm.at[wid, pl.ds(hp * HALF, HALF)], src_v)
                pltpu.sync_copy(dst_hbm.at[wid, pl.ds(hp * HALF, HALF)], dst_v)
                gath(0, rows_a, sem_a)
                gath(1, rows_b, sem_b)

            def body(p, carry):
                j = 2 * p
                scat(j, rows_a, sem_a)
                gath(j + 2, rows_a, sem_a)
                scat(j + 1, rows_b, sem_b)
                gath(j + 3, rows_b, sem_b)
                return carry

            lax.fori_loop(0, HALF // 2 - 1, body, 0)
            scat(HALF - 2, rows_a, sem_a)
            scat(HALF - 1, rows_b, sem_b)
        plsc.subcore_barrier()
        pltpu.sync_copy(table.at[pl.ds(r0, RPT)], out_hbm.at[c, pl.ds(r0, RPT)])

    return _sc_prop


def _sc_deg(*args):
    return _build_sc_deg()(*args)


def _sc_prop(*args):
    return _build_sc_prop()(*args)


def _rowmask(i):
    rows = lax.broadcasted_iota(jnp.int32, (BLK, 1), 0) + i * BLK
    return rows < N


def _dot(a, b):
    return jnp.dot(a, b, preferred_element_type=jnp.float32,
                   precision=lax.Precision.HIGHEST)


def _tcgate_body(x_ref, wg_ref, bg_ref, h0_ref):
    i = pl.program_id(0)
    h = jax.nn.sigmoid(_dot(x_ref[...], wg_ref[...]) + bg_ref[...])
    h0_ref[...] = jnp.where(_rowmask(i), h, 0.0)


def _tcnorm_body(degp_ref, h0_ref, hs0_ref, norm_ref):
    deg = degp_ref[0, :, 0:1] + degp_ref[1, :, 0:1]
    norm = lax.rsqrt(jnp.maximum(deg, 1.0))
    hs0_ref[...] = h0_ref[...] * norm
    norm_ref[...] = jnp.broadcast_to(norm, (BLK, 128))


def _tcscale_body(norm_ref, ap_ref, f_ref, fs_ref):
    norm = norm_ref[...]
    a = ap_ref[0] + ap_ref[1]
    f = a * norm
    f_ref[...] = f
    fs_ref[...] = f * norm


def _tcpre_body(h_ref, f1_ref, w_ref, b_ref, pc_ref):
    w = w_ref[...]
    pc_ref[...] = _dot(h_ref[...], w[0:128]) + _dot(f1_ref[...], w[128:256]) \
        + b_ref[...]


def _tcconv_body(norm_ref, ap_ref, pc_ref, w_ref, t_ref, ts_ref):
    i = pl.program_id(0)
    norm = norm_ref[...]
    f2 = (ap_ref[0] + ap_ref[1]) * norm
    t = pc_ref[...] + _dot(f2, w_ref[256:384])
    t = jnp.maximum(t, 0.0)
    t = jnp.where(_rowmask(i), t, 0.0)
    t_ref[...] = t
    ts_ref[...] = t * norm


def _tcfinal_body(norm_ref, ap_ref, pc_ref, wc_ref,
                  wd_ref, bd_ref, we1_ref, be1_ref, we21_ref, be21_ref,
                  we22_ref, be22_ref, wdc1_ref, bdc1_ref, wdc21_ref,
                  bdc21_ref, wdc22_ref, bdc22_ref, eps1_ref, eps2_ref,
                  zsum_ref, zest_ref, lrec_ref, kld_ref):
    i = pl.program_id(0)
    norm = norm_ref[...]
    g2 = (ap_ref[0] + ap_ref[1]) * norm
    t2 = pc_ref[...] + _dot(g2, wc_ref[256:384])
    t2 = jnp.maximum(t2, 0.0)
    hfin = _dot(t2, wd_ref[...]) + bd_ref[...]
    hfin = jnp.where(_rowmask(i), hfin, 0.0)

    @pl.when(i == 0)
    def _():
        zsum_ref[...] = jnp.zeros_like(zsum_ref)

    zsum_ref[...] += jnp.sum(hfin, axis=0, keepdims=True)

    @pl.when(i == GRID - 1)
    def _():
        z = zsum_ref[...] * (1.0 / N)
        h1 = jnp.maximum(_dot(z, we1_ref[...]) + be1_ref[...], 0.0)
        mu = _dot(h1, we21_ref[...]) + be21_ref[...]
        logvar = _dot(h1, we22_ref[...]) + be22_ref[...]
        latent = mu + eps1_ref[...] * jnp.exp(0.5 * logvar)
        h2 = jnp.maximum(_dot(latent, wdc1_ref[...]) + bdc1_ref[...], 0.0)
        z_mu = _dot(h2, wdc21_ref[...]) + bdc21_ref[...]
        z_logvar = _dot(h2, wdc22_ref[...]) + bdc22_ref[...]
        zest_ref[...] = z_mu + eps2_ref[...] * jnp.exp(0.5 * z_logvar)
        lrec_ref[...] = (LOG2PI + z_logvar
                         + (z - z_mu) ** 2 / (2.0 * jnp.exp(z_logvar)))
        kld_ref[...] = jnp.full((1, 128), -0.5 * jnp.sum(
            1.0 + logvar - mu ** 2 - jnp.exp(logvar)))


def _blk(ix=None):
    if ix is None:
        return pl.BlockSpec((BLK, 128), lambda i: (i, 0))
    return pl.BlockSpec(ix[0], ix[1])


_DEGP_SPEC = pl.BlockSpec((2, BLK, 128), lambda i: (0, i, 0))
_AP_SPEC = pl.BlockSpec((2, BLK, 128), lambda i: (0, i, 0))
_W128_SPEC = pl.BlockSpec((128, 128), lambda i: (0, 0))
_W384_SPEC = pl.BlockSpec((384, 128), lambda i: (0, 0))
_B_SPEC = pl.BlockSpec((1, 128), lambda i: (0, 0))
_NP_F32 = jax.ShapeDtypeStruct((NP, 128), jnp.float32)


def _tc_gate(x, wg, bg):
    return pl.pallas_call(
        _tcgate_body,
        grid=(GRID,),
        in_specs=[_blk(), _W128_SPEC, _B_SPEC],
        out_specs=_blk(),
        out_shape=_NP_F32,
    )(x, wg, bg)


def _tc_norm(degp, h0):
    return pl.pallas_call(
        _tcnorm_body,
        grid=(GRID,),
        in_specs=[_DEGP_SPEC, _blk()],
        out_specs=[_blk(), _blk()],
        out_shape=[_NP_F32, _NP_F32],
    )(degp, h0)


def _tc_scale(norm, ap):
    return pl.pallas_call(
        _tcscale_body,
        grid=(GRID,),
        in_specs=[_blk(), _AP_SPEC],
        out_specs=[_blk(), _blk()],
        out_shape=[_NP_F32, _NP_F32],
    )(norm, ap)


def _tc_pre(h, f1, w, b):
    return pl.pallas_call(
        _tcpre_body,
        grid=(GRID,),
        in_specs=[_blk(), _blk(), _W384_SPEC, _B_SPEC],
        out_specs=_blk(),
        out_shape=_NP_F32,
    )(h, f1, w, b)


def _tc_conv(norm, ap, pc, w):
    return pl.pallas_call(
        _tcconv_body,
        grid=(GRID,),
        in_specs=[_blk(), _AP_SPEC, _blk(), _W384_SPEC],
        out_specs=[_blk(), _blk()],
        out_shape=[_NP_F32, _NP_F32],
    )(norm, ap, pc, w)


def _tc_final(norm, ap, pc, wc, wd, bd, *vae_ws):
    return pl.pallas_call(
        _tcfinal_body,
        grid=(GRID,),
        in_specs=[_blk(), _AP_SPEC, _blk(), _W384_SPEC,
                  _W128_SPEC, _B_SPEC]
                 + [_W128_SPEC, _B_SPEC] * 6 + [_B_SPEC, _B_SPEC],
        out_specs=[pl.BlockSpec((1, 128), lambda i: (0, 0))] * 4,
        out_shape=[jax.ShapeDtypeStruct((1, 128), jnp.float32)] * 4,
    )(norm, ap, pc, wc, wd, bd, *vae_ws)


def _pad_w(w, b, fin, fout):
    wp = jnp.zeros((128, 128), jnp.float32).at[:fin, :fout].set(w)
    bp = jnp.zeros((1, 128), jnp.float32).at[0, :fout].set(b)
    return wp, bp


def kernel(x, edge_index, W_gate, b_gate, W_c0, b_c0, W_c1, b_c1, W_d, b_d,
           W_e1, b_e1, W_e21, b_e21, W_e22, b_e22,
           W_dc1, b_dc1, W_dc21, b_dc21, W_dc22, b_dc22):
    f32 = jnp.float32
    pad = ZROW + jnp.arange(EP - E, dtype=jnp.int32) % (NP - N)
    src_p = jnp.concatenate([edge_index[0], pad]).reshape(NTILES, CH, 128)
    dst_p = jnp.concatenate([edge_index[1], pad]).reshape(NTILES, CH, 128)
    zeros128 = jnp.zeros((NP, 128), f32)
    ones128 = jnp.ones((128, 128), f32)

    degp = _sc_deg(dst_p, zeros128, ones128)
    h0 = _tc_gate(x, W_gate, b_gate.reshape(1, 128))
    hs0, norm = _tc_norm(degp, h0)
    a1p = _sc_prop(src_p, dst_p, hs0, zeros128)
    f1, f1s = _tc_scale(norm, a1p)
    a2p = _sc_prop(src_p, dst_p, f1s, zeros128)
    pc1 = _tc_pre(h0, f1, W_c0, b_c0.reshape(1, 128))
    t1, t1s = _tc_conv(norm, a2p, pc1, W_c0)
    b1p = _sc_prop(src_p, dst_p, t1s, zeros128)
    g1, g1s = _tc_scale(norm, b1p)
    b2p = _sc_prop(src_p, dst_p, g1s, zeros128)
    pc2 = _tc_pre(t1, g1, W_c1, b_c1.reshape(1, 128))

    we1p, be1p = _pad_w(W_e1, b_e1, 128, 64)
    we21p, be21p = _pad_w(W_e21, b_e21, 64, 32)
    we22p, be22p = _pad_w(W_e22, b_e22, 64, 32)
    wdc1p, bdc1p = _pad_w(W_dc1, b_dc1, 32, 64)
    wdc21p, bdc21p = _pad_w(W_dc21, b_dc21, 64, 128)
    wdc22p, bdc22p = _pad_w(W_dc22, b_dc22, 64, 128)
    eps1 = jnp.zeros((1, 128), f32).at[0, :32].set(
        jax.random.normal(jax.random.key(1), (32,), f32))
    eps2 = jax.random.normal(jax.random.key(2), (128,), f32).reshape(1, 128)

    _, zest, lrec, kld = _tc_final(
        norm, b2p, pc2, W_c1, W_d, b_d.reshape(1, 128),
        we1p, be1p, we21p, be21p, we22p, be22p,
        wdc1p, bdc1p, wdc21p, bdc21p, wdc22p, bdc22p, eps1, eps2)
    return (zest.reshape(128), lrec.reshape(128), kld[0, 0])
